# Initial kernel scaffold; baseline (speedup 1.0000x reference)
import jax
import jax.numpy as jnp
from jax.experimental import pallas as pl


def kernel(edge_index, user_embedding):
    raise NotImplementedError("write your pallas kernel here")


# speedup vs baseline: 10.7839x; 10.7839x over previous
import jax
import jax.numpy as jnp
from jax import lax
from jax.experimental import pallas as pl
from jax.experimental.pallas import tpu as pltpu
from jax.experimental.pallas import tpu_sc as plsc

N = 10000
D = 128
E = 320000
NC = 2
NS = 16
NW = NC * NS
C = 128
EPW = E // NW
KS = (EPW + C - 1) // C
KG = KS + 1
PADW = KG * C - EPW
NPAD = 10240
RPT = NPAD // NS

_MESH = plsc.VectorSubcoreMesh(core_axis_name="c", subcore_axis_name="s")


def _deg_body(col_hbm, ones_hbm, zeros_hbm, out_hbm, col_v, ones_v, acc):
    c = lax.axis_index("c")
    s = lax.axis_index("s")
    w = c * NS + s
    pltpu.sync_copy(zeros_hbm, acc.at[pl.ds(s * RPT, RPT)])
    pltpu.sync_copy(ones_hbm, ones_v)
    pltpu.sync_copy(col_hbm.at[w], col_v)
    plsc.subcore_barrier()

    def step(j, carry):
        pltpu.sync_copy(ones_v, acc.at[col_v.at[j]], add=True)
        return carry

    lax.fori_loop(0, KS, step, 0)
    plsc.subcore_barrier()
    pltpu.sync_copy(acc.at[pl.ds(s * RPT, RPT)],
                    out_hbm.at[c, pl.ds(s * RPT, RPT)])


_deg_call = pl.kernel(
    _deg_body,
    out_type=jax.ShapeDtypeStruct((NC, NPAD), jnp.float32),
    mesh=_MESH,
    scratch_types=[
        pltpu.VMEM((KG, C), jnp.int32),
        pltpu.VMEM((C,), jnp.float32),
        pltpu.VMEM_SHARED((NPAD,), jnp.float32),
    ],
)


def _layer_body(tbl_hbm, row_hbm, col_hbm, zeros_hbm, out_hbm,
                row_v, col_v, buf, sem, acc):
    c = lax.axis_index("c")
    s = lax.axis_index("s")
    w = c * NS + s
    pltpu.sync_copy(zeros_hbm, acc.at[pl.ds(s * RPT, RPT)])
    pltpu.sync_copy(row_hbm.at[w], row_v)
    pltpu.sync_copy(col_hbm.at[w], col_v)
    plsc.subcore_barrier()

    def step(j, carry):
        pltpu.async_copy(tbl_hbm.at[row_v.at[j]], buf, sem).wait()
        pltpu.sync_copy(buf, acc.at[col_v.at[j]], add=True)
        return carry

    lax.fori_loop(0, KS, step, 0)
    plsc.subcore_barrier()
    pltpu.sync_copy(acc.at[pl.ds(s * RPT, RPT)],
                    out_hbm.at[c, pl.ds(s * RPT, RPT)])


_layer_call = pl.kernel(
    _layer_body,
    out_type=jax.ShapeDtypeStruct((NC, NPAD, D), jnp.float32),
    mesh=_MESH,
    scratch_types=[
        pltpu.VMEM((KG, C), jnp.int32),
        pltpu.VMEM((KG, C), jnp.int32),
        pltpu.VMEM((C, D), jnp.float32),
        pltpu.SemaphoreType.DMA,
        pltpu.VMEM_SHARED((NPAD, D), jnp.float32),
    ],
)


def _dinv_of(deg_ref):
    d = deg_ref[0] + deg_ref[1]
    return jnp.where(d > 0, lax.rsqrt(d), 0.0)


def _stage1_body(deg_ref, e_ref, u_ref):
    u_ref[...] = _dinv_of(deg_ref) * e_ref[...]


def _stage2_body(deg_ref, s_ref, x1_ref, v_ref):
    dinv = _dinv_of(deg_ref)
    x1 = dinv * (s_ref[0] + s_ref[1])
    x1_ref[...] = x1
    v_ref[...] = dinv * x1


def _stage3_body(deg_ref, e_ref, x1_ref, s_ref, o_ref):
    x2 = _dinv_of(deg_ref) * (s_ref[0] + s_ref[1])
    o_ref[...] = (e_ref[...] + x1_ref[...] + x2) * (1.0 / 3.0)


_f32 = jnp.float32
_stage1 = pl.pallas_call(
    _stage1_body, out_shape=jax.ShapeDtypeStruct((NPAD, D), _f32))
_stage2 = pl.pallas_call(
    _stage2_body, out_shape=(jax.ShapeDtypeStruct((NPAD, D), _f32),
                             jax.ShapeDtypeStruct((NPAD, D), _f32)))
_stage3 = pl.pallas_call(
    _stage3_body, out_shape=jax.ShapeDtypeStruct((NPAD, D), _f32))


def kernel(edge_index, user_embedding):
    ei = edge_index.astype(jnp.int32)
    r = ei[0].reshape(NW, EPW)
    co = ei[1].reshape(NW, EPW)
    rp = jnp.pad(r, ((0, 0), (0, PADW)))
    cp = jnp.pad(co, ((0, 0), (0, PADW)), constant_values=N)
    row_blk = rp.reshape(NW, KG, C)
    col_blk = cp.reshape(NW, KG, C)
    tbl = jnp.pad(user_embedding, ((0, NPAD - N), (0, 0)))
    ones = jnp.ones((C,), _f32)
    zer1 = jnp.zeros((RPT,), _f32)
    zer2 = jnp.zeros((RPT, D), _f32)

    deg2 = _deg_call(col_blk, ones, zer1).reshape(NC, NPAD, 1)
    u = _stage1(deg2, tbl)
    s1 = _layer_call(u, row_blk, col_blk, zer2)
    x1, v = _stage2(deg2, s1)
    s2 = _layer_call(v, row_blk, col_blk, zer2)
    out = _stage3(deg2, tbl, x1, s2)
    return out[:N]
